# baseline (speedup 1.0000x reference)
import functools

import jax
import jax.numpy as jnp
from jax import lax
from jax.experimental import pallas as pl
from jax.experimental.pallas import tpu as pltpu
from jax.experimental.pallas import tpu_sc as plsc

BATCH = 16384
D = 32
NC = 2
NS = 16
L = 16
NW = NC * NS
CHUNK = BATCH // NW
GROUPS = CHUNK // L


def _sc_body(user_emb, user_ids, item_emb, item_ids, out,
             uidx_v, iidx_v, urows_v, irows_v, preds_v, sem):
    wid = lax.axis_index("s") * NC + lax.axis_index("c")
    base = wid * CHUNK

    pltpu.sync_copy(user_ids.at[pl.ds(base, CHUNK)], uidx_v)
    pltpu.sync_copy(item_ids.at[pl.ds(base, CHUNK)], iidx_v)

    cu = pltpu.async_copy(user_emb.at[uidx_v], urows_v, sem)
    ci = pltpu.async_copy(item_emb.at[iidx_v], irows_v, sem)
    cu.wait()
    ci.wait()

    lane = lax.broadcasted_iota(jnp.int32, (L,), 0)
    last = lane == (L - 1)

    def row(i, carry):
        u0 = urows_v[i, pl.ds(0, L)]
        u1 = urows_v[i, pl.ds(L, L)]
        v0 = irows_v[i, pl.ds(0, L)]
        v1 = irows_v[i, pl.ds(L, L)]
        s = plsc.cumsum(u0 * v0 + u1 * v1)
        plsc.store_scatter(preds_v, [jnp.full((L,), i, jnp.int32)], s,
                           mask=last)
        return carry

    lax.fori_loop(0, CHUNK, row, 0)

    pltpu.sync_copy(preds_v, out.at[pl.ds(base, CHUNK)])


_sc_preds = pl.kernel(
    _sc_body,
    out_type=jax.ShapeDtypeStruct((BATCH,), jnp.float32),
    mesh=plsc.VectorSubcoreMesh(core_axis_name="c", subcore_axis_name="s"),
    compiler_params=pltpu.CompilerParams(
        needs_layout_passes=False, use_tc_tiling_on_sc=False
    ),
    scratch_types=[
        pltpu.VMEM((CHUNK,), jnp.int32),
        pltpu.VMEM((CHUNK,), jnp.int32),
        pltpu.VMEM((CHUNK, D), jnp.float32),
        pltpu.VMEM((CHUNK, D), jnp.float32),
        pltpu.VMEM((CHUNK,), jnp.float32),
        pltpu.SemaphoreType.DMA,
    ],
)


def _loss_body(p_ref, r_ref, o_ref):
    d = p_ref[...] - r_ref[...]
    o_ref[0, 0] = jnp.sum(d * d) / BATCH


_loss = pl.pallas_call(
    _loss_body,
    out_shape=jax.ShapeDtypeStruct((1, 1), jnp.float32),
    out_specs=pl.BlockSpec(memory_space=pltpu.SMEM),
)


@jax.jit
def kernel(user_ids, item_ids, ratings, user_emb, item_emb):
    preds = _sc_preds(user_emb, user_ids, item_emb, item_ids)
    loss = _loss(preds.reshape(128, 128), ratings.reshape(128, 128))[0, 0]
    return preds, loss


# speedup vs baseline: 1.4908x; 1.4908x over previous
import functools

import jax
import jax.numpy as jnp
from jax import lax
from jax.experimental import pallas as pl
from jax.experimental.pallas import tpu as pltpu
from jax.experimental.pallas import tpu_sc as plsc

BATCH = 16384
D = 32
NC = 2
NS = 16
L = 16
NW = NC * NS
CHUNK = BATCH // NW
HALF = CHUNK // 2


def _sc_body(user_emb, user_ids, item_emb, item_ids, out,
             uidx_v, iidx_v, urows_v, irows_v, preds_v, sem):
    wid = lax.axis_index("s") * NC + lax.axis_index("c")
    base = wid * CHUNK

    pltpu.sync_copy(user_ids.at[pl.ds(base, CHUNK)], uidx_v)
    pltpu.sync_copy(item_ids.at[pl.ds(base, CHUNK)], iidx_v)

    lane = lax.broadcasted_iota(jnp.int32, (L,), 0)
    last = lane == (L - 1)

    for h in range(2):
        hbase = h * HALF

        def issue(g, carry):
            uvec = uidx_v[pl.ds(hbase + g * L, L)]
            ivec = iidx_v[pl.ds(hbase + g * L, L)]
            for j in range(L):
                i = g * L + j
                pltpu.async_copy(user_emb.at[uvec[j]], urows_v.at[i], sem)
                pltpu.async_copy(item_emb.at[ivec[j]], irows_v.at[i], sem)
            return carry

        lax.fori_loop(0, HALF // L, issue, 0)

        def drain(g, carry):
            uvec = uidx_v[pl.ds(hbase + g * L, L)]
            ivec = iidx_v[pl.ds(hbase + g * L, L)]
            for j in range(L):
                i = g * L + j
                pltpu.make_async_copy(
                    user_emb.at[uvec[j]], urows_v.at[i], sem).wait()
                pltpu.make_async_copy(
                    item_emb.at[ivec[j]], irows_v.at[i], sem).wait()
            return carry

        lax.fori_loop(0, HALF // L, drain, 0)

        def row(i, carry):
            u0 = urows_v[i, pl.ds(0, L)]
            u1 = urows_v[i, pl.ds(L, L)]
            v0 = irows_v[i, pl.ds(0, L)]
            v1 = irows_v[i, pl.ds(L, L)]
            s = plsc.cumsum(u0 * v0 + u1 * v1)
            plsc.store_scatter(
                preds_v, [jnp.full((L,), hbase + i, jnp.int32)], s, mask=last)
            return carry

        lax.fori_loop(0, HALF, row, 0)

    pltpu.sync_copy(preds_v, out.at[pl.ds(base, CHUNK)])


_sc_preds = pl.kernel(
    _sc_body,
    out_type=jax.ShapeDtypeStruct((BATCH,), jnp.float32),
    mesh=plsc.VectorSubcoreMesh(core_axis_name="c", subcore_axis_name="s"),
    compiler_params=pltpu.CompilerParams(needs_layout_passes=False),
    scratch_types=[
        pltpu.VMEM((CHUNK,), jnp.int32),
        pltpu.VMEM((CHUNK,), jnp.int32),
        pltpu.VMEM((HALF, D), jnp.float32),
        pltpu.VMEM((HALF, D), jnp.float32),
        pltpu.VMEM((CHUNK,), jnp.float32),
        pltpu.SemaphoreType.DMA,
    ],
)


def _loss_body(p_ref, r_ref, o_ref):
    d = p_ref[...] - r_ref[...]
    o_ref[0, 0] = jnp.sum(d * d) / BATCH


_loss = pl.pallas_call(
    _loss_body,
    out_shape=jax.ShapeDtypeStruct((1, 1), jnp.float32),
    out_specs=pl.BlockSpec(memory_space=pltpu.SMEM),
)


@jax.jit
def kernel(user_ids, item_ids, ratings, user_emb, item_emb):
    preds = _sc_preds(user_emb, user_ids, item_emb, item_ids)
    loss = _loss(preds.reshape(128, 128), ratings.reshape(128, 128))[0, 0]
    return preds, loss
